# baseline (speedup 1.0000x reference)
import functools

import jax
import jax.numpy as jnp
from jax import lax
from jax.experimental import pallas as pl
from jax.experimental.pallas import tpu as pltpu
from jax.experimental.pallas import tpu_sc as plsc

N = 10000
E = 160000
U = 32
MU = 32
DE = 16
BN_EPS = 1e-3

NC, NS = 2, 16
NW = NC * NS
W = 128
WPW = 40
EP = NW * WPW * W
NWIN = EP // W
PAD = EP - E
NSH = 10240
RPT = NSH // NS
DUMMY = N + 16
NSLAB = 2
EPS = EP // NSLAB
NWIN_S = NWIN // NSLAB

_vmesh = plsc.VectorSubcoreMesh(core_axis_name="c", subcore_axis_name="s")


@functools.partial(
    pl.kernel,
    mesh=_vmesh,
    out_type=jax.ShapeDtypeStruct((EPS, 128), jnp.float32),
    scratch_types=[
        pltpu.VMEM_SHARED((NSH, 128), jnp.float32),
    ],
)
def _gather_kernel(table_hbm, idx_hbm, out_hbm, table_sh):
    sid = lax.axis_index("s")
    pltpu.sync_copy(table_hbm.at[pl.ds(sid * RPT, RPT)],
                    table_sh.at[pl.ds(sid * RPT, RPT)])
    plsc.subcore_barrier()

    def body(i_vmem, o_vmem):
        pltpu.sync_copy(table_sh.at[i_vmem.at[0]], o_vmem)

    pltpu.emit_pipeline(
        body,
        grid=(NWIN_S,),
        in_specs=[pl.BlockSpec((1, W), lambda i: (0, i))],
        out_specs=[pl.BlockSpec((W, 128), lambda i: (i, 0))],
        core_axis_name=("c", "s"),
        dimension_semantics=(pltpu.PARALLEL,),
    )(idx_hbm, out_hbm)


@functools.partial(
    pl.kernel,
    mesh=_vmesh,
    out_type=jax.ShapeDtypeStruct((NC * NSH, 128), jnp.float32),
    scratch_types=[
        pltpu.VMEM_SHARED((NSH, 128), jnp.float32),
    ],
)
def _scatter_kernel(msg0, msg1, dst_hbm, zeros_hbm, sums_hbm, sums_sh):
    cid = lax.axis_index("c")
    sid = lax.axis_index("s")

    pltpu.sync_copy(zeros_hbm, sums_sh.at[pl.ds(sid * RPT, RPT)])
    plsc.subcore_barrier()

    def body(m_vmem, d_vmem):
        pltpu.sync_copy(m_vmem, sums_sh.at[d_vmem.at[0]], add=True)

    for s, msg_hbm in enumerate((msg0, msg1)):
        pltpu.emit_pipeline(
            body,
            grid=(NWIN_S,),
            in_specs=[
                pl.BlockSpec((W, 128), lambda i: (i, 0)),
                pl.BlockSpec((1, W), lambda i, s=s: (0, s * NWIN_S + i)),
            ],
            out_specs=[],
            core_axis_name=("c", "s"),
            dimension_semantics=(pltpu.PARALLEL,),
        )(msg_hbm, dst_hbm)

    plsc.subcore_barrier()
    pltpu.sync_copy(sums_sh.at[pl.ds(sid * RPT, RPT)],
                    sums_hbm.at[pl.ds(cid * NSH + sid * RPT, RPT)])


BE = 4096


def _msg_body(ef_ref, src_ref, w1_ref, b1_ref, rh_ref, w2g_ref, b2t_ref,
              out_ref):
    src = src_ref[:, :U].astype(jnp.bfloat16)
    h = jnp.maximum(ef_ref[...] @ w1_ref[...] + b1_ref[...], 0.0)
    g2 = jax.lax.dot(src, w2g_ref[...],
                     preferred_element_type=jnp.float32)
    hrep = jax.lax.dot(h.astype(jnp.bfloat16), rh_ref[...],
                       preferred_element_type=jnp.float32)
    p = hrep * g2
    t = [p[:, 128 * a:128 * (a + 1)] for a in range(8)]
    acc = ((t[0] + t[1]) + (t[2] + t[3])) + ((t[4] + t[5]) + (t[6] + t[7]))
    msg = (acc[:, 0:32] + acc[:, 32:64]) + (acc[:, 64:96] + acc[:, 96:128])
    out_ref[:, :U] = msg + jax.lax.dot(src, b2t_ref[...],
                                       preferred_element_type=jnp.float32)
    out_ref[:, U:2 * U] = jnp.ones((BE, U), jnp.float32)


def _compute_messages(ef, src, w1, b1, rh, w2g, b2t):
    const = lambda shape: pl.BlockSpec(shape, lambda i: (0, 0))
    return pl.pallas_call(
        _msg_body,
        grid=(EPS // BE,),
        in_specs=[
            pl.BlockSpec((BE, DE), lambda i: (i, 0)),
            pl.BlockSpec((BE, 128), lambda i: (i, 0)),
            const((DE, U)),
            const((1, U)),
            const((U, U * MU)),
            const((U, U * MU)),
            const((U, U)),
        ],
        out_specs=pl.BlockSpec((BE, 128), lambda i: (i, 0)),
        out_shape=jax.ShapeDtypeStruct((EPS, 128), jnp.float32),
    )(ef, src, w1, b1, rh, w2g, b2t)


BNODE = 2000


def _final_body(sums_ref, ns_ref, gk_ref, grk_ref, gbi_ref,
                gbr_ref, gamma_ref, beta_ref, mean_ref, var_ref,
                out_ref, hnew_ref):
    both = sums_ref[0] + sums_ref[1]
    s = both[:, :U]
    c = both[:, U:U + 1]
    agg = s / jnp.maximum(c, 1.0)
    ns = ns_ref[...]
    xm = agg @ gk_ref[...] + gbi_ref[...]
    rm = ns @ grk_ref[...] + gbr_ref[...]
    z = jax.nn.sigmoid(xm[:, :U] + rm[:, :U])
    r = jax.nn.sigmoid(xm[:, U:2 * U] + rm[:, U:2 * U])
    hh = jnp.tanh(xm[:, 2 * U:] + r * rm[:, 2 * U:])
    h_new = z * ns + (1.0 - z) * hh
    o = gamma_ref[...] * (h_new - mean_ref[...]) * jax.lax.rsqrt(
        var_ref[...] + BN_EPS) + beta_ref[...]
    out_ref[...] = jnp.tanh(o)
    hnew_ref[...] = h_new


def _finalize(sums_p, node_state, gk, grk, gbi, gbr,
              gamma, beta, mean, var):
    const = lambda shape: pl.BlockSpec(shape, lambda i: (0, 0))
    return pl.pallas_call(
        _final_body,
        grid=(N // BNODE,),
        in_specs=[
            pl.BlockSpec((NC, BNODE, 128), lambda i: (0, i, 0)),
            pl.BlockSpec((BNODE, U), lambda i: (i, 0)),
            const((U, 3 * U)),
            const((U, 3 * U)),
            const((1, 3 * U)),
            const((1, 3 * U)),
            const((1, U)),
            const((1, U)),
            const((1, U)),
            const((1, U)),
        ],
        out_specs=[
            pl.BlockSpec((BNODE, U), lambda i: (i, 0)),
            pl.BlockSpec((BNODE, U), lambda i: (i, 0)),
        ],
        out_shape=[
            jax.ShapeDtypeStruct((N, U), jnp.float32),
            jax.ShapeDtypeStruct((N, U), jnp.float32),
        ],
    )(sums_p, node_state, gk, grk, gbi, gbr, gamma, beta, mean, var)


def kernel(node_features, edge_features, node_state, w1, b1, w2, b2,
           gru_k, gru_rk, gru_bi, gru_br, bn_gamma, bn_beta, bn_mean, bn_var,
           edge_indices):
    src_idx = jnp.concatenate(
        [edge_indices[0], jnp.zeros((PAD,), jnp.int32)]).reshape(1, EP)
    dst_idx = jnp.concatenate(
        [edge_indices[1], jnp.full((PAD,), DUMMY, jnp.int32)]
    ).reshape(1, EP)
    ef_p = jnp.pad(edge_features, ((0, PAD), (0, 0)))

    rh = jnp.repeat(jnp.eye(U, dtype=jnp.bfloat16), U, axis=1)
    w2g = w2.reshape(U, U, MU).transpose(2, 0, 1).reshape(MU, U * U)
    w2g = w2g.astype(jnp.bfloat16)
    b2t = b2.reshape(U, MU).T.astype(jnp.bfloat16)

    nf_p = jnp.pad(node_features, ((0, NSH - N), (0, 128 - U)))
    srcs = [_gather_kernel(nf_p, src_idx[:, s * EPS:(s + 1) * EPS])
            for s in range(NSLAB)]
    msgs = [_compute_messages(ef_p[s * EPS:(s + 1) * EPS], srcs[s],
                              w1, b1.reshape(1, U), rh, w2g, b2t)
            for s in range(NSLAB)]

    zeros = jnp.zeros((RPT, 128), jnp.float32)
    sums_p = _scatter_kernel(*msgs, dst_idx, zeros).reshape(NC, NSH, 128)

    return _finalize(sums_p, node_state, gru_k, gru_rk,
                     gru_bi.reshape(1, 3 * U), gru_br.reshape(1, 3 * U),
                     bn_gamma.reshape(1, U), bn_beta.reshape(1, U),
                     bn_mean.reshape(1, U), bn_var.reshape(1, U))


# speedup vs baseline: 1.0006x; 1.0006x over previous
import functools

import jax
import jax.numpy as jnp
from jax import lax
from jax.experimental import pallas as pl
from jax.experimental.pallas import tpu as pltpu
from jax.experimental.pallas import tpu_sc as plsc

N = 10000
E = 160000
U = 32
MU = 32
DE = 16
BN_EPS = 1e-3

NC, NS = 2, 16
NW = NC * NS
W = 128
WPW = 40
EP = NW * WPW * W
NWIN = EP // W
PAD = EP - E
NSH = 10240
RPT = NSH // NS
DUMMY = N + 16
NSLAB = 1
EPS = EP // NSLAB
NWIN_S = NWIN // NSLAB

_vmesh = plsc.VectorSubcoreMesh(core_axis_name="c", subcore_axis_name="s")


@functools.partial(
    pl.kernel,
    mesh=_vmesh,
    out_type=jax.ShapeDtypeStruct((EPS, 128), jnp.float32),
    scratch_types=[
        pltpu.VMEM_SHARED((NSH, 128), jnp.float32),
    ],
)
def _gather_kernel(table_hbm, idx_hbm, out_hbm, table_sh):
    sid = lax.axis_index("s")
    pltpu.sync_copy(table_hbm.at[pl.ds(sid * RPT, RPT)],
                    table_sh.at[pl.ds(sid * RPT, RPT)])
    plsc.subcore_barrier()

    def body(i_vmem, o_vmem):
        pltpu.sync_copy(table_sh.at[i_vmem.at[0]], o_vmem)

    pltpu.emit_pipeline(
        body,
        grid=(NWIN_S,),
        in_specs=[pl.BlockSpec((1, W), lambda i: (0, i))],
        out_specs=[pl.BlockSpec((W, 128), lambda i: (i, 0))],
        core_axis_name=("c", "s"),
        dimension_semantics=(pltpu.PARALLEL,),
    )(idx_hbm, out_hbm)


@functools.partial(
    pl.kernel,
    mesh=_vmesh,
    out_type=jax.ShapeDtypeStruct((NC * NSH, 128), jnp.float32),
    scratch_types=[
        pltpu.VMEM_SHARED((NSH, 128), jnp.float32),
    ],
)
def _scatter_kernel(msg0, dst_hbm, zeros_hbm, sums_hbm, sums_sh):
    cid = lax.axis_index("c")
    sid = lax.axis_index("s")

    pltpu.sync_copy(zeros_hbm, sums_sh.at[pl.ds(sid * RPT, RPT)])
    plsc.subcore_barrier()

    def body(m_vmem, d_vmem):
        pltpu.sync_copy(m_vmem, sums_sh.at[d_vmem.at[0]], add=True)

    for s, msg_hbm in enumerate((msg0,)):
        pltpu.emit_pipeline(
            body,
            grid=(NWIN_S,),
            in_specs=[
                pl.BlockSpec((W, 128), lambda i: (i, 0)),
                pl.BlockSpec((1, W), lambda i, s=s: (0, s * NWIN_S + i)),
            ],
            out_specs=[],
            core_axis_name=("c", "s"),
            dimension_semantics=(pltpu.PARALLEL,),
        )(msg_hbm, dst_hbm)

    plsc.subcore_barrier()
    pltpu.sync_copy(sums_sh.at[pl.ds(sid * RPT, RPT)],
                    sums_hbm.at[pl.ds(cid * NSH + sid * RPT, RPT)])


BE = 4096


def _msg_body(ef_ref, src_ref, w1_ref, b1_ref, rh_ref, w2g_ref, b2t_ref,
              out_ref):
    src = src_ref[:, :U].astype(jnp.bfloat16)
    h = jnp.maximum(ef_ref[...] @ w1_ref[...] + b1_ref[...], 0.0)
    g2 = jax.lax.dot(src, w2g_ref[...],
                     preferred_element_type=jnp.float32)
    hrep = jax.lax.dot(h.astype(jnp.bfloat16), rh_ref[...],
                       preferred_element_type=jnp.float32)
    p = hrep * g2
    t = [p[:, 128 * a:128 * (a + 1)] for a in range(8)]
    acc = ((t[0] + t[1]) + (t[2] + t[3])) + ((t[4] + t[5]) + (t[6] + t[7]))
    msg = (acc[:, 0:32] + acc[:, 32:64]) + (acc[:, 64:96] + acc[:, 96:128])
    out_ref[:, :U] = msg + jax.lax.dot(src, b2t_ref[...],
                                       preferred_element_type=jnp.float32)
    out_ref[:, U:2 * U] = jnp.ones((BE, U), jnp.float32)


def _compute_messages(ef, src, w1, b1, rh, w2g, b2t):
    const = lambda shape: pl.BlockSpec(shape, lambda i: (0, 0))
    return pl.pallas_call(
        _msg_body,
        grid=(EPS // BE,),
        in_specs=[
            pl.BlockSpec((BE, DE), lambda i: (i, 0)),
            pl.BlockSpec((BE, 128), lambda i: (i, 0)),
            const((DE, U)),
            const((1, U)),
            const((U, U * MU)),
            const((U, U * MU)),
            const((U, U)),
        ],
        out_specs=pl.BlockSpec((BE, 128), lambda i: (i, 0)),
        out_shape=jax.ShapeDtypeStruct((EPS, 128), jnp.float32),
    )(ef, src, w1, b1, rh, w2g, b2t)


BNODE = 2000


def _final_body(sums_ref, ns_ref, gk_ref, grk_ref, gbi_ref,
                gbr_ref, gamma_ref, beta_ref, mean_ref, var_ref,
                out_ref, hnew_ref):
    both = sums_ref[0] + sums_ref[1]
    s = both[:, :U]
    c = both[:, U:U + 1]
    agg = s / jnp.maximum(c, 1.0)
    ns = ns_ref[...]
    xm = agg @ gk_ref[...] + gbi_ref[...]
    rm = ns @ grk_ref[...] + gbr_ref[...]
    z = jax.nn.sigmoid(xm[:, :U] + rm[:, :U])
    r = jax.nn.sigmoid(xm[:, U:2 * U] + rm[:, U:2 * U])
    hh = jnp.tanh(xm[:, 2 * U:] + r * rm[:, 2 * U:])
    h_new = z * ns + (1.0 - z) * hh
    o = gamma_ref[...] * (h_new - mean_ref[...]) * jax.lax.rsqrt(
        var_ref[...] + BN_EPS) + beta_ref[...]
    out_ref[...] = jnp.tanh(o)
    hnew_ref[...] = h_new


def _finalize(sums_p, node_state, gk, grk, gbi, gbr,
              gamma, beta, mean, var):
    const = lambda shape: pl.BlockSpec(shape, lambda i: (0, 0))
    return pl.pallas_call(
        _final_body,
        grid=(N // BNODE,),
        in_specs=[
            pl.BlockSpec((NC, BNODE, 128), lambda i: (0, i, 0)),
            pl.BlockSpec((BNODE, U), lambda i: (i, 0)),
            const((U, 3 * U)),
            const((U, 3 * U)),
            const((1, 3 * U)),
            const((1, 3 * U)),
            const((1, U)),
            const((1, U)),
            const((1, U)),
            const((1, U)),
        ],
        out_specs=[
            pl.BlockSpec((BNODE, U), lambda i: (i, 0)),
            pl.BlockSpec((BNODE, U), lambda i: (i, 0)),
        ],
        out_shape=[
            jax.ShapeDtypeStruct((N, U), jnp.float32),
            jax.ShapeDtypeStruct((N, U), jnp.float32),
        ],
    )(sums_p, node_state, gk, grk, gbi, gbr, gamma, beta, mean, var)


def kernel(node_features, edge_features, node_state, w1, b1, w2, b2,
           gru_k, gru_rk, gru_bi, gru_br, bn_gamma, bn_beta, bn_mean, bn_var,
           edge_indices):
    src_idx = jnp.concatenate(
        [edge_indices[0], jnp.zeros((PAD,), jnp.int32)]).reshape(1, EP)
    dst_idx = jnp.concatenate(
        [edge_indices[1], jnp.full((PAD,), DUMMY, jnp.int32)]
    ).reshape(1, EP)
    ef_p = jnp.pad(edge_features, ((0, PAD), (0, 0)))

    rh = jnp.repeat(jnp.eye(U, dtype=jnp.bfloat16), U, axis=1)
    w2g = w2.reshape(U, U, MU).transpose(2, 0, 1).reshape(MU, U * U)
    w2g = w2g.astype(jnp.bfloat16)
    b2t = b2.reshape(U, MU).T.astype(jnp.bfloat16)

    nf_p = jnp.pad(node_features, ((0, NSH - N), (0, 128 - U)))
    srcs = [_gather_kernel(nf_p, src_idx[:, s * EPS:(s + 1) * EPS])
            for s in range(NSLAB)]
    msgs = [_compute_messages(ef_p[s * EPS:(s + 1) * EPS], srcs[s],
                              w1, b1.reshape(1, U), rh, w2g, b2t)
            for s in range(NSLAB)]

    zeros = jnp.zeros((RPT, 128), jnp.float32)
    sums_p = _scatter_kernel(*msgs, dst_idx, zeros).reshape(NC, NSH, 128)

    return _finalize(sums_p, node_state, gru_k, gru_rk,
                     gru_bi.reshape(1, 3 * U), gru_br.reshape(1, 3 * U),
                     bn_gamma.reshape(1, U), bn_beta.reshape(1, U),
                     bn_mean.reshape(1, U), bn_var.reshape(1, U))
